# Initial kernel scaffold; baseline (speedup 1.0000x reference)
import jax
import jax.numpy as jnp
from jax.experimental import pallas as pl


def kernel(r_ij, edge_index, W_a, W_v, W_d):
    raise NotImplementedError("write your pallas kernel here")


# speedup vs baseline: 40.6461x; 40.6461x over previous
import functools

import jax
import jax.numpy as jnp
from jax import lax
from jax.experimental import pallas as pl
from jax.experimental.pallas import tpu as pltpu
from jax.experimental.pallas import tpu_sc as plsc

N_NODES = 50000
N_EDGES = 800000
R0 = 5.0
NRAD = 8
DIM_A, DIM_V, DIM_D = 64, 32, 16

NCHUNK = 8
CW = 16
EB = 3200
N_PAD = 50176
NB = 3136
SC_CORES = 2
SC_TILES = 16
E_PER_TILE = N_EDGES // SC_TILES
WIN = 2000
N_WINS = E_PER_TILE // WIN
N_STRIPE = N_PAD // SC_TILES


def _phi_body(rt_ref, out_ref):
    r = rt_ref[...]
    x = r[0:1, :]
    y = r[1:2, :]
    z = r[2:3, :]
    x_sq = (x * x + y * y + z * z) * (1.0 / R0)
    s = jnp.sqrt(x_sq)
    env = jnp.maximum(1.0 - x_sq, 0.0)
    c = lax.broadcasted_iota(jnp.int32, (NRAD, 1), 0).astype(jnp.float32)
    rad = jnp.cos((jnp.pi * c) * s) * env

    v = r * (17.0 / R0)
    nrm = jnp.sqrt((v * v).sum(axis=0, keepdims=True) + 1e-9)
    vs = v * (jax.nn.sigmoid(nrm) / nrm)

    ms = [jnp.ones_like(x_sq), vs[0:1, :], vs[1:2, :], vs[2:3, :]]
    for i in range(3):
        for j in range(3):
            ms.append(vs[i:i + 1, :] * vs[j:j + 1, :])

    zero8 = jnp.zeros((NRAD, EB), jnp.float32)
    for q in range(NCHUNK):
        rows = []
        for k in range(2 * q, 2 * q + 2):
            rows.append(rad * ms[k] if k < len(ms) else zero8)
        chunk = jnp.concatenate(rows, axis=0)
        out_ref[q] = chunk.T


def _phi_pallas(rt):
    return pl.pallas_call(
        _phi_body,
        grid=(N_EDGES // EB,),
        in_specs=[pl.BlockSpec((3, EB), lambda i: (0, i))],
        out_specs=pl.BlockSpec((NCHUNK, EB, CW), lambda i: (0, i, 0)),
        out_shape=jax.ShapeDtypeStruct((NCHUNK, N_EDGES, CW), jnp.float32),
    )(rt)


def _scatter_body(src_hbm, phi_hbm, zeros_hbm, a_hbm, idx_v, pay_v, acc_sh):
    cid = lax.axis_index("c")
    sid = lax.axis_index("s")

    def run_pass(q):
        pltpu.sync_copy(zeros_hbm.at[pl.ds(sid * N_STRIPE, N_STRIPE)],
                        acc_sh.at[pl.ds(sid * N_STRIPE, N_STRIPE)])
        plsc.subcore_barrier()

        @pl.loop(0, N_WINS)
        def _(w):
            base = sid * E_PER_TILE + w * WIN
            pltpu.sync_copy(src_hbm.at[pl.ds(base, WIN)], idx_v)
            pltpu.sync_copy(phi_hbm.at[q, pl.ds(base, WIN)], pay_v)
            pltpu.sync_copy(pay_v, acc_sh.at[idx_v], add=True)

        plsc.subcore_barrier()
        pltpu.sync_copy(acc_sh.at[pl.ds(sid * N_STRIPE, N_STRIPE)],
                        a_hbm.at[q, pl.ds(sid * N_STRIPE, N_STRIPE)])

    for j in range(NCHUNK // SC_CORES):
        run_pass(cid * (NCHUNK // SC_CORES) + j)


def _segment_sum_sc(src, phi, zeros):
    mesh = plsc.VectorSubcoreMesh(core_axis_name="c", subcore_axis_name="s",
                                  num_cores=SC_CORES, num_subcores=SC_TILES)
    kern = pl.kernel(
        _scatter_body,
        out_type=jax.ShapeDtypeStruct((NCHUNK, N_PAD, CW), jnp.float32),
        mesh=mesh,
        scratch_types=[
            pltpu.VMEM((WIN,), jnp.int32),
            pltpu.VMEM((WIN, CW), jnp.float32),
            pltpu.VMEM_SHARED((N_PAD, CW), jnp.float32),
        ],
        compiler_params=pltpu.CompilerParams(use_tc_tiling_on_sc=False),
    )
    return kern(src, phi, zeros)


def _matmul_body(a_ref, w_ref, out_ref):
    acc = jnp.zeros((NB, 304), jnp.float32)
    for q in range(NCHUNK):
        acc = acc + lax.dot_general(
            a_ref[q], w_ref[pl.ds(q * CW, CW), :],
            (((1,), (0,)), ((), ())),
            preferred_element_type=jnp.float32)
    out_ref[...] = acc


def _matmul_pallas(a, wbig):
    return pl.pallas_call(
        _matmul_body,
        grid=(N_PAD // NB,),
        in_specs=[pl.BlockSpec((NCHUNK, NB, CW), lambda i: (0, i, 0)),
                  pl.BlockSpec((NCHUNK * CW, 304), lambda i: (0, 0))],
        out_specs=pl.BlockSpec((NB, 304), lambda i: (i, 0)),
        out_shape=jax.ShapeDtypeStruct((N_PAD, 304), jnp.float32),
    )(a, wbig)


def _build_wbig(w_a, w_v, w_d):
    wbig = jnp.zeros((NCHUNK * CW, 304), jnp.float32)
    wbig = wbig.at[0:NRAD, 0:DIM_A].set(w_a)
    for t in range(3):
        wbig = wbig.at[NRAD * (1 + t):NRAD * (2 + t), 64 + t:160:3].set(w_v)
    for i in range(3):
        for j in range(3):
            k = 4 + 3 * i + j
            wbig = wbig.at[NRAD * k:NRAD * (k + 1), 160 + 3 * i + j:304:9].set(w_d)
    return wbig


def kernel(r_ij, edge_index, W_a, W_v, W_d):
    src = edge_index[0]
    rt = r_ij.T
    phi = _phi_pallas(rt)
    zeros = jnp.zeros((N_PAD, CW), jnp.float32)
    a = _segment_sum_sc(src, phi, zeros)
    wbig = _build_wbig(W_a, W_v, W_d)
    b = _matmul_pallas(a, wbig)[:N_NODES]
    b_a = b[:, :DIM_A]
    b_v = b[:, 64:160].reshape(N_NODES, DIM_V, 3)
    b_d = b[:, 160:304].reshape(N_NODES, DIM_D, 3, 3)
    return (b_a, b_v, b_d)


# speedup vs baseline: 107.7742x; 2.6515x over previous
import functools

import jax
import jax.numpy as jnp
from jax import lax
from jax.experimental import pallas as pl
from jax.experimental.pallas import tpu as pltpu
from jax.experimental.pallas import tpu_sc as plsc

N_NODES = 50000
N_EDGES = 800000
R0 = 5.0
NRAD = 8
DIM_A, DIM_V, DIM_D = 64, 32, 16

NCHUNK = 8
CW = 16
EB = 3200
N_PAD = 50176
NB = 3136
SC_CORES = 2
SC_TILES = 16
E_PER_TILE = N_EDGES // SC_TILES
WIN = 2000
N_WINS = E_PER_TILE // WIN
N_STRIPE = N_PAD // SC_TILES


def _phi_body(rt_ref, out_ref):
    r = rt_ref[...]
    x = r[0:1, :]
    y = r[1:2, :]
    z = r[2:3, :]
    x_sq = (x * x + y * y + z * z) * (1.0 / R0)
    s = jnp.sqrt(x_sq)
    env = jnp.maximum(1.0 - x_sq, 0.0)
    c = lax.broadcasted_iota(jnp.int32, (NRAD, 1), 0).astype(jnp.float32)
    rad = jnp.cos((jnp.pi * c) * s) * env

    v = r * (17.0 / R0)
    nrm = jnp.sqrt((v * v).sum(axis=0, keepdims=True) + 1e-9)
    vs = v * (jax.nn.sigmoid(nrm) / nrm)

    ms = [jnp.ones_like(x_sq), vs[0:1, :], vs[1:2, :], vs[2:3, :]]
    for i in range(3):
        for j in range(3):
            ms.append(vs[i:i + 1, :] * vs[j:j + 1, :])

    zero8 = jnp.zeros((NRAD, EB), jnp.float32)
    rows = [rad * ms[k] if k < len(ms) else zero8 for k in range(16)]
    phi = jnp.concatenate(rows, axis=0)
    out_ref[...] = phi.T


def _phi_pallas(rt):
    return pl.pallas_call(
        _phi_body,
        grid=(N_EDGES // EB,),
        in_specs=[pl.BlockSpec((3, EB), lambda i: (0, i))],
        out_specs=pl.BlockSpec((EB, 128), lambda i: (i, 0)),
        out_shape=jax.ShapeDtypeStruct((N_EDGES, 128), jnp.float32),
    )(rt)


def _scatter_body(src_hbm, phi_hbm, zeros_hbm, a_hbm, idx_v, pay_v, acc_sh):
    cid = lax.axis_index("c")
    sid = lax.axis_index("s")

    def run_pass(q):
        pltpu.sync_copy(zeros_hbm,
                        acc_sh.at[pl.ds(sid * N_STRIPE, N_STRIPE)])
        plsc.subcore_barrier()

        @pl.loop(0, N_WINS)
        def _(w):
            base = sid * E_PER_TILE + w * WIN
            pltpu.sync_copy(src_hbm.at[pl.ds(base, WIN)], idx_v)
            pltpu.sync_copy(phi_hbm.at[pl.ds(base, WIN), pl.ds(q * CW, CW)],
                            pay_v)
            pltpu.sync_copy(pay_v, acc_sh.at[idx_v], add=True)

        plsc.subcore_barrier()
        pltpu.sync_copy(acc_sh.at[pl.ds(sid * N_STRIPE, N_STRIPE)],
                        a_hbm.at[pl.ds(sid * N_STRIPE, N_STRIPE),
                                 pl.ds(q * CW, CW)])

    for j in range(NCHUNK // SC_CORES):
        run_pass(cid * (NCHUNK // SC_CORES) + j)


def _segment_sum_sc(src, phi, zeros):
    mesh = plsc.VectorSubcoreMesh(core_axis_name="c", subcore_axis_name="s",
                                  num_cores=SC_CORES, num_subcores=SC_TILES)
    kern = pl.kernel(
        _scatter_body,
        out_type=jax.ShapeDtypeStruct((N_PAD, 128), jnp.float32),
        mesh=mesh,
        scratch_types=[
            pltpu.VMEM((WIN,), jnp.int32),
            pltpu.VMEM((WIN, CW), jnp.float32),
            pltpu.VMEM_SHARED((N_PAD, CW), jnp.float32),
        ],
        compiler_params=pltpu.CompilerParams(use_tc_tiling_on_sc=False),
    )
    return kern(src, phi, zeros)


def _matmul_body(a_ref, w_ref, out_ref):
    out_ref[...] = lax.dot_general(
        a_ref[...], w_ref[...],
        (((1,), (0,)), ((), ())),
        preferred_element_type=jnp.float32)


def _matmul_pallas(a, wbig):
    return pl.pallas_call(
        _matmul_body,
        grid=(N_PAD // NB,),
        in_specs=[pl.BlockSpec((NB, 128), lambda i: (i, 0)),
                  pl.BlockSpec((NCHUNK * CW, 304), lambda i: (0, 0))],
        out_specs=pl.BlockSpec((NB, 304), lambda i: (i, 0)),
        out_shape=jax.ShapeDtypeStruct((N_PAD, 304), jnp.float32),
    )(a, wbig)


def _build_wbig(w_a, w_v, w_d):
    wbig = jnp.zeros((NCHUNK * CW, 304), jnp.float32)
    wbig = wbig.at[0:NRAD, 0:DIM_A].set(w_a)
    for t in range(3):
        wbig = wbig.at[NRAD * (1 + t):NRAD * (2 + t), 64 + t:160:3].set(w_v)
    for i in range(3):
        for j in range(3):
            k = 4 + 3 * i + j
            wbig = wbig.at[NRAD * k:NRAD * (k + 1), 160 + 3 * i + j:304:9].set(w_d)
    return wbig


def kernel(r_ij, edge_index, W_a, W_v, W_d):
    src = edge_index[0]
    rt = r_ij.T
    phi = _phi_pallas(rt)
    zeros = jnp.zeros((N_STRIPE, CW), jnp.float32)
    a = _segment_sum_sc(src, phi, zeros)
    wbig = _build_wbig(W_a, W_v, W_d)
    b = _matmul_pallas(a, wbig)[:N_NODES]
    b_a = b[:, :DIM_A]
    b_v = b[:, 64:160].reshape(N_NODES, DIM_V, 3)
    b_d = b[:, 160:304].reshape(N_NODES, DIM_D, 3, 3)
    return (b_a, b_v, b_d)
